# baseline (speedup 1.0000x reference)
import functools

import jax
import jax.numpy as jnp
from jax import lax
from jax.experimental import pallas as pl
from jax.experimental.pallas import tpu as pltpu
from jax.experimental.pallas import tpu_sc as plsc

E = 8
TOPK = 2
NEG = -1e30
T = 2048
D = 1024
DFF = 512
BLK = 256
NB = T * TOPK // BLK + E
NPAD = NB * BLK
CH = 512


def _silu(v):
    return v * (1.0 / (1.0 + jnp.exp(-v)))


def _dot_t(a, b):
    return jax.lax.dot_general(a, b, (((1,), (1,)), ((), ())),
                               preferred_element_type=jnp.float32)


def _dot(a, b):
    return jax.lax.dot_general(a, b, (((1,), (0,)), ((), ())),
                               preferred_element_type=jnp.float32)


def _plan_body(x_ref, wr_ref, posw_ref, be_ref):
    x = x_ref[...]

    lg = _dot_t(x, wr_ref[...])
    lanes = jax.lax.broadcasted_iota(jnp.int32, (T, 128), 1)
    valid = lanes < E
    lg = jnp.where(valid, lg, NEG)
    m1 = jnp.max(lg, axis=1, keepdims=True)
    i1 = jnp.min(jnp.where(lg == m1, lanes, 999), axis=1, keepdims=True)
    lg2 = jnp.where(lanes == i1, NEG, lg)
    m2 = jnp.max(lg2, axis=1, keepdims=True)
    i2 = jnp.min(jnp.where(lg2 == m2, lanes, 999), axis=1, keepdims=True)
    d = jnp.exp(m2 - m1)
    w1 = 1.0 / (1.0 + d)
    w2 = 1.0 - w1

    oh1 = jnp.where(lanes == i1, 1.0, 0.0)
    oh2 = jnp.where(lanes == i2, 1.0, 0.0)

    r = jax.lax.broadcasted_iota(jnp.int32, (CH, CH), 0)
    c = jax.lax.broadcasted_iota(jnp.int32, (CH, CH), 1)
    tril = jnp.where(r > c, 1.0, 0.0)
    ones_row = jnp.ones((1, CH), jnp.float32)
    chunks1 = []
    chunks2 = []
    off1 = jnp.zeros((1, 128), jnp.float32)
    off2 = jnp.zeros((1, 128), jnp.float32)
    for b in range(T // CH):
        s = slice(b * CH, (b + 1) * CH)
        c1 = oh1[s, :]
        c2 = oh2[s, :]
        chunks1.append(_dot(tril, c1) + off1)
        chunks2.append(_dot(tril, c2) + off2)
        off1 = off1 + _dot(ones_row, c1)
        off2 = off2 + _dot(ones_row, c2)
    exc1 = jnp.concatenate(chunks1, axis=0)
    exc2 = jnp.concatenate(chunks2, axis=0)
    tot1 = off1
    counts = off1 + off2

    fblk = jnp.float32(BLK)
    nblocks = jnp.floor((counts + (fblk - 1.0)) * (1.0 / fblk))
    padded = nblocks * fblk

    r128 = jax.lax.broadcasted_iota(jnp.int32, (128, 128), 0)
    c128 = jax.lax.broadcasted_iota(jnp.int32, (128, 128), 1)
    ustrict = jnp.where(r128 < c128, 1.0, 0.0)
    lstrict = jnp.where(r128 > c128, 1.0, 0.0)
    starts_row = _dot(padded, ustrict)
    starts_col = jax.lax.dot_general(
        lstrict, padded, (((1,), (1,)), ((), ())),
        preferred_element_type=jnp.float32)
    nblocks_col = jax.lax.dot_general(
        jnp.where(r128 == c128, 1.0, 0.0), nblocks, (((1,), (1,)), ((), ())),
        preferred_element_type=jnp.float32)

    sb_col = starts_col * (1.0 / fblk)
    eb_col = sb_col + nblocks_col
    bcols = jax.lax.broadcasted_iota(jnp.int32, (128, 128), 1).astype(
        jnp.float32)
    erows = jax.lax.broadcasted_iota(jnp.int32, (128, 128), 0)
    cmp = jnp.where((sb_col <= bcols) & (erows < E), 1.0, 0.0)
    be_row = _dot(jnp.ones((1, 128), jnp.float32), cmp) - 1.0
    be_row = jnp.clip(be_row, 0.0, float(E - 1))
    nb_used = jnp.sum(nblocks, axis=1, keepdims=True)
    lane1 = jax.lax.broadcasted_iota(jnp.int32, (1, 128), 1)
    be_out = jnp.where(lane1 == 127, nb_used, be_row)
    be_ref[...] = be_out.astype(jnp.int32)

    pos1 = jnp.sum(oh1 * (starts_row + exc1), axis=1, keepdims=True)
    pos2 = jnp.sum(oh2 * (starts_row + tot1 + exc2), axis=1, keepdims=True)

    posw = (jnp.where(lane1 == 0, 1.0, 0.0) * pos1
            + jnp.where(lane1 == 1, 1.0, 0.0) * pos2
            + jnp.where(lane1 == 2, 1.0, 0.0) * w1
            + jnp.where(lane1 == 3, 1.0, 0.0) * w2)
    posw_ref[...] = posw


def _plan(x, wr_p):
    return pl.pallas_call(
        _plan_body,
        grid=(1,),
        in_specs=[
            pl.BlockSpec((T, D), lambda i: (0, 0)),
            pl.BlockSpec((128, D), lambda i: (0, 0)),
        ],
        out_specs=[
            pl.BlockSpec((T, 128), lambda i: (0, 0)),
            pl.BlockSpec((1, 128), lambda i: (0, 0)),
        ],
        out_shape=[
            jax.ShapeDtypeStruct((T, 128), jnp.float32),
            jax.ShapeDtypeStruct((1, 128), jnp.int32),
        ],
    )(x, wr_p)


def _ffn_body(be_ref, xs_ref, wg_ref, wu_ref, wd_ref, ys_ref):
    b = pl.program_id(0)
    nb = be_ref[127]

    @pl.when(b < nb)
    def _():
        xs = xs_ref[...]
        h = _silu(_dot_t(xs, wg_ref[0])) * _dot_t(xs, wu_ref[0])
        ys_ref[...] = _dot_t(h, wd_ref[0])


def _ffn(be, xs, W_gate, W_up, W_down):
    grid_spec = pltpu.PrefetchScalarGridSpec(
        num_scalar_prefetch=1,
        grid=(NB,),
        in_specs=[
            pl.BlockSpec((BLK, D), lambda b, be: (b, 0)),
            pl.BlockSpec((1, DFF, D), lambda b, be: (be[b], 0, 0)),
            pl.BlockSpec((1, DFF, D), lambda b, be: (be[b], 0, 0)),
            pl.BlockSpec((1, D, DFF), lambda b, be: (be[b], 0, 0)),
        ],
        out_specs=pl.BlockSpec((BLK, D), lambda b, be: (b, 0)),
    )
    return pl.pallas_call(
        _ffn_body,
        grid_spec=grid_spec,
        out_shape=jax.ShapeDtypeStruct((NPAD, D), jnp.float32),
    )(be, xs, W_gate, W_up, W_down)


_NC = 2
_NW = 32
_TPW = T // _NW
_SCH = 32
_CCH = 16


@functools.lru_cache(maxsize=1)
def _sc_kernels():
    mesh = plsc.VectorSubcoreMesh(core_axis_name="c", subcore_axis_name="s")

    @functools.partial(
        pl.kernel,
        mesh=mesh,
        out_type=jax.ShapeDtypeStruct((NPAD, D), jnp.float32),
        scratch_types=[
            pltpu.VMEM((_SCH, D), jnp.float32),
            pltpu.VMEM((_SCH,), jnp.int32),
            pltpu.VMEM((_SCH,), jnp.int32),
            pltpu.SemaphoreType.DMA,
        ],
    )
    def scatter_sc(x_hbm, pos1_hbm, pos2_hbm, xs_hbm, xbuf, idx1, idx2,
                   sem):
        wid = lax.axis_index("s") * _NC + lax.axis_index("c")
        base = wid * _TPW
        for ci in range(_TPW // _SCH):
            b0 = base + ci * _SCH
            pltpu.sync_copy(pos1_hbm.at[pl.ds(b0, _SCH)], idx1)
            pltpu.sync_copy(pos2_hbm.at[pl.ds(b0, _SCH)], idx2)
            pltpu.sync_copy(x_hbm.at[pl.ds(b0, _SCH)], xbuf)
            pltpu.async_copy(xbuf, xs_hbm.at[idx1], sem).wait()
            pltpu.async_copy(xbuf, xs_hbm.at[idx2], sem).wait()

    @functools.partial(
        pl.kernel,
        mesh=mesh,
        out_type=jax.ShapeDtypeStruct((T, D), jnp.float32),
        scratch_types=[
            pltpu.VMEM((_CCH, D), jnp.float32),
            pltpu.VMEM((_CCH, D), jnp.float32),
            pltpu.VMEM((_CCH, D), jnp.float32),
            pltpu.VMEM((_CCH,), jnp.int32),
            pltpu.VMEM((_CCH,), jnp.int32),
            pltpu.VMEM((_CCH,), jnp.float32),
            pltpu.VMEM((_CCH,), jnp.float32),
            pltpu.SemaphoreType.DMA,
            pltpu.SemaphoreType.DMA,
        ],
    )
    def combine_sc(ys_hbm, pos1_hbm, pos2_hbm, w1_hbm, w2_hbm, out_hbm,
                   buf1, buf2, obuf, idx1, idx2, w1v, w2v, sem1, sem2):
        wid = lax.axis_index("s") * _NC + lax.axis_index("c")
        base = wid * _TPW
        for ci in range(_TPW // _CCH):
            b0 = base + ci * _CCH
            pltpu.sync_copy(pos1_hbm.at[pl.ds(b0, _CCH)], idx1)
            pltpu.sync_copy(pos2_hbm.at[pl.ds(b0, _CCH)], idx2)
            pltpu.sync_copy(w1_hbm.at[pl.ds(b0, _CCH)], w1v)
            pltpu.sync_copy(w2_hbm.at[pl.ds(b0, _CCH)], w2v)
            cp1 = pltpu.async_copy(ys_hbm.at[idx1], buf1, sem1)
            cp2 = pltpu.async_copy(ys_hbm.at[idx2], buf2, sem2)
            cp1.wait()
            cp2.wait()

            w1vec = w1v[...]
            w2vec = w2v[...]
            aa = [w1vec[t] for t in range(_CCH)]
            bb = [w2vec[t] for t in range(_CCH)]

            def lanechunk(j, carry):
                sl = pl.ds(j * 16, 16)
                for t in range(_CCH):
                    obuf[t, sl] = aa[t] * buf1[t, sl] + bb[t] * buf2[t, sl]
                return carry

            lax.fori_loop(0, D // 16, lanechunk, 0)
            pltpu.sync_copy(obuf, out_hbm.at[pl.ds(b0, _CCH)])

    return scatter_sc, combine_sc


def kernel(hidden_states, W_router, W_gate, W_up, W_down):
    b, s, d = hidden_states.shape
    x = hidden_states.reshape(T, d)
    wr_p = jnp.zeros((128, d), jnp.float32).at[:E].set(W_router)

    posw, be = _plan(x, wr_p)
    pos1 = posw[:, 0].astype(jnp.int32)
    pos2 = posw[:, 1].astype(jnp.int32)
    w1 = posw[:, 2]
    w2 = posw[:, 3]

    scatter_sc, combine_sc = _sc_kernels()
    xs = scatter_sc(x, pos1, pos2)
    ys = _ffn(be.reshape(128), xs, W_gate, W_up, W_down)
    out = combine_sc(ys, pos1, pos2, w1, w2)
    return out.reshape(b, s, d)
